# baseline (speedup 1.0000x reference)
import functools

import jax
import jax.numpy as jnp
from jax import lax
from jax.experimental import pallas as pl
from jax.experimental.pallas import tpu as pltpu
from jax.experimental.pallas import tpu_sc as plsc

T_TABLE = 1001
EMBED_DIM = 128
BATCH = 16384

NC = 2
NS = 16
NW = NC * NS
CHUNK = 128
NCHUNK = BATCH // NW // CHUNK
B_PER_W = BATCH // NW


def _sc_gather(table, idx):
    mesh = plsc.VectorSubcoreMesh(core_axis_name="c", subcore_axis_name="s")

    @functools.partial(
        pl.kernel,
        out_type=jax.ShapeDtypeStruct((BATCH, EMBED_DIM), jnp.float32),
        mesh=mesh,
        scratch_types=[
            pltpu.VMEM((NCHUNK, CHUNK), jnp.int32),
            pltpu.VMEM((NCHUNK, CHUNK, EMBED_DIM), jnp.float32),
            pltpu.SemaphoreType.DMA((NCHUNK,)),
            pltpu.SemaphoreType.DMA((NCHUNK,)),
            pltpu.SemaphoreType.DMA((NCHUNK,)),
        ],
    )
    def k(table_hbm, idx_hbm, out_hbm, idx_v, rows_v, isem, gsem, osem):
        wid = lax.axis_index("s") * NC + lax.axis_index("c")
        base = wid * B_PER_W
        idx_cps = [
            pltpu.async_copy(
                idx_hbm.at[pl.ds(base + j * CHUNK, CHUNK)], idx_v.at[j], isem.at[j]
            )
            for j in range(NCHUNK)
        ]
        gathers = []
        for j in range(NCHUNK):
            idx_cps[j].wait()
            gathers.append(
                pltpu.async_copy(table_hbm.at[idx_v.at[j]], rows_v.at[j], gsem.at[j])
            )
        outs = []
        for j in range(NCHUNK):
            gathers[j].wait()
            outs.append(
                pltpu.async_copy(
                    rows_v.at[j], out_hbm.at[pl.ds(base + j * CHUNK, CHUNK)], osem.at[j]
                )
            )
        for o in outs:
            o.wait()

    return k(table, idx)


def kernel(t, time_encodings):
    return _sc_gather(time_encodings, t.astype(jnp.int32))
